# baseline (speedup 1.0000x reference)
import functools

import jax
import jax.numpy as jnp
from jax import lax
from jax.experimental import pallas as pl
from jax.experimental.pallas import tpu as pltpu
from jax.experimental.pallas import tpu_sc as plsc

BATCH = 1024
CTX = 20
EMBED = 16
VOCAB = 100000

NC, NS = 2, 16
NW = NC * NS
B_PER_W = BATCH // NW
ELEM_PER_W = B_PER_W * CTX * EMBED
IDX_CHUNK = 128
N_CHUNKS = ELEM_PER_W // IDX_CHUNK
FIRE = 16
N_ROUNDS = N_CHUNKS // FIRE


def _gather_mean_sc(eidx, table_flat):
    mesh = plsc.VectorSubcoreMesh(core_axis_name="c", subcore_axis_name="s")

    @functools.partial(
        pl.kernel,
        mesh=mesh,
        compiler_params=pltpu.CompilerParams(use_tc_tiling_on_sc=False),
        out_type=jax.ShapeDtypeStruct((BATCH, EMBED), jnp.float32),
        scratch_types=[
            pltpu.VMEM((ELEM_PER_W,), jnp.int32),
            pltpu.VMEM((ELEM_PER_W,), jnp.float32),
            pltpu.VMEM((B_PER_W, EMBED), jnp.float32),
            pltpu.SemaphoreType.DMA,
        ],
    )
    def gather_mean(eidx_hbm, table_hbm, avg_hbm, idx_v, rows_v, avg_v, sem):
        wid = lax.axis_index("s") * NC + lax.axis_index("c")
        pltpu.sync_copy(eidx_hbm.at[wid], idx_v)
        pltpu.async_copy(table_hbm.at[idx_v], rows_v, sem).wait()

        inv_ctx = jnp.float32(1.0 / CTX)

        def body(i, carry):
            base = i * (CTX * EMBED)
            acc = rows_v[pl.ds(base, EMBED)]
            for j in range(1, CTX):
                acc = acc + rows_v[pl.ds(base + j * EMBED, EMBED)]
            avg_v[i] = acc * inv_ctx
            return carry

        lax.fori_loop(0, B_PER_W, body, 0)
        pltpu.sync_copy(avg_v, avg_hbm.at[pl.ds(wid * B_PER_W, B_PER_W)])

    return gather_mean(eidx, table_flat)


VT = 4096


def _mm_body(w_ref, avgt_ref, b_ref, out_ref):
    out_ref[...] = (
        lax.dot_general(
            w_ref[...],
            avgt_ref[...],
            (((0,), (0,)), ((), ())),
            preferred_element_type=jnp.float32,
        )
        + b_ref[...][:, None]
    )


def _matmul_bias_tc(avg_t, fc_w, fc_b):
    nvt = (VOCAB + VT - 1) // VT
    out_t = pl.pallas_call(
        _mm_body,
        grid=(nvt,),
        in_specs=[
            pl.BlockSpec((EMBED, VT), lambda j: (0, j)),
            pl.BlockSpec((EMBED, BATCH), lambda j: (0, 0)),
            pl.BlockSpec((VT,), lambda j: (j,)),
        ],
        out_specs=pl.BlockSpec((VT, BATCH), lambda j: (j, 0)),
        out_shape=jax.ShapeDtypeStruct((VOCAB, BATCH), jnp.float32),
    )(fc_w, avg_t, fc_b)
    return jnp.transpose(out_t)


def kernel(x, emb_table, fc_w, fc_b):
    eidx = (
        x.reshape(-1, 1).astype(jnp.int32)
        + jnp.arange(EMBED, dtype=jnp.int32)[None, :] * VOCAB
    ).reshape(NW, ELEM_PER_W)
    table_flat = emb_table.T.reshape(-1)
    avg = _gather_mean_sc(eidx, table_flat)
    return _matmul_bias_tc(avg.T, fc_w, fc_b)


# speedup vs baseline: 1.0006x; 1.0006x over previous
import functools

import jax
import jax.numpy as jnp
from jax import lax
from jax.experimental import pallas as pl
from jax.experimental.pallas import tpu as pltpu
from jax.experimental.pallas import tpu_sc as plsc

BATCH = 1024
CTX = 20
EMBED = 16
VOCAB = 100000

NC, NS = 2, 16
NW = NC * NS
B_PER_W = BATCH // NW
ELEM_PER_W = B_PER_W * CTX * EMBED
IDX_CHUNK = 128
N_CHUNKS = ELEM_PER_W // IDX_CHUNK
FIRE = 16
N_ROUNDS = N_CHUNKS // FIRE


def _gather_mean_sc(eidx, table_flat):
    mesh = plsc.VectorSubcoreMesh(core_axis_name="c", subcore_axis_name="s")

    @functools.partial(
        pl.kernel,
        mesh=mesh,
        compiler_params=pltpu.CompilerParams(use_tc_tiling_on_sc=False),
        out_type=jax.ShapeDtypeStruct((BATCH, EMBED), jnp.float32),
        scratch_types=[
            pltpu.VMEM((ELEM_PER_W,), jnp.int32),
            pltpu.VMEM((ELEM_PER_W,), jnp.float32),
            pltpu.VMEM((B_PER_W, EMBED), jnp.float32),
            pltpu.SemaphoreType.DMA,
        ],
    )
    def gather_mean(eidx_hbm, table_hbm, avg_hbm, idx_v, rows_v, avg_v, sem):
        wid = lax.axis_index("s") * NC + lax.axis_index("c")
        pltpu.sync_copy(eidx_hbm.at[wid], idx_v)
        for k in range(16):
            pltpu.async_copy(
                table_hbm.at[idx_v.at[pl.ds(k * 640, 640)]],
                rows_v.at[pl.ds(k * 640, 640)],
                sem,
            )
        pltpu.make_async_copy(
            table_hbm.at[pl.ds(0, ELEM_PER_W)], rows_v, sem
        ).wait()

        inv_ctx = jnp.float32(1.0 / CTX)

        def body(i, carry):
            base = i * (CTX * EMBED)
            acc = rows_v[pl.ds(base, EMBED)]
            for j in range(1, CTX):
                acc = acc + rows_v[pl.ds(base + j * EMBED, EMBED)]
            avg_v[i] = acc * inv_ctx
            return carry

        lax.fori_loop(0, B_PER_W, body, 0)
        pltpu.sync_copy(avg_v, avg_hbm.at[pl.ds(wid * B_PER_W, B_PER_W)])

    return gather_mean(eidx, table_flat)


VT = 4096


def _mm_body(w_ref, avgt_ref, b_ref, out_ref):
    out_ref[...] = (
        lax.dot_general(
            w_ref[...],
            avgt_ref[...],
            (((0,), (0,)), ((), ())),
            preferred_element_type=jnp.float32,
        )
        + b_ref[...][:, None]
    )


def _matmul_bias_tc(avg_t, fc_w, fc_b):
    nvt = (VOCAB + VT - 1) // VT
    out_t = pl.pallas_call(
        _mm_body,
        grid=(nvt,),
        in_specs=[
            pl.BlockSpec((EMBED, VT), lambda j: (0, j)),
            pl.BlockSpec((EMBED, BATCH), lambda j: (0, 0)),
            pl.BlockSpec((VT,), lambda j: (j,)),
        ],
        out_specs=pl.BlockSpec((VT, BATCH), lambda j: (j, 0)),
        out_shape=jax.ShapeDtypeStruct((VOCAB, BATCH), jnp.float32),
    )(fc_w, avg_t, fc_b)
    return jnp.transpose(out_t)


def kernel(x, emb_table, fc_w, fc_b):
    eidx = (
        x.reshape(-1, 1).astype(jnp.int32)
        + jnp.arange(EMBED, dtype=jnp.int32)[None, :] * VOCAB
    ).reshape(NW, ELEM_PER_W)
    table_flat = emb_table.T.reshape(-1)
    avg = _gather_mean_sc(eidx, table_flat)
    return _matmul_bias_tc(avg.T, fc_w, fc_b)


# speedup vs baseline: 1.0234x; 1.0227x over previous
import functools

import jax
import jax.numpy as jnp
from jax import lax
from jax.experimental import pallas as pl
from jax.experimental.pallas import tpu as pltpu
from jax.experimental.pallas import tpu_sc as plsc

BATCH = 1024
CTX = 20
EMBED = 16
VOCAB = 100000

NC, NS = 2, 16
NW = NC * NS
B_PER_W = BATCH // NW
ELEM_PER_W = B_PER_W * CTX * EMBED
IDX_CHUNK = 128
N_CHUNKS = ELEM_PER_W // IDX_CHUNK
FIRE = 16
N_ROUNDS = N_CHUNKS // FIRE


def _gather_mean_sc(eidx, table_flat):
    mesh = plsc.VectorSubcoreMesh(core_axis_name="c", subcore_axis_name="s")

    @functools.partial(
        pl.kernel,
        mesh=mesh,
        compiler_params=pltpu.CompilerParams(use_tc_tiling_on_sc=False),
        out_type=jax.ShapeDtypeStruct((BATCH, EMBED), jnp.float32),
        scratch_types=[
            pltpu.VMEM((N_CHUNKS, IDX_CHUNK), jnp.int32),
            pltpu.VMEM((ELEM_PER_W,), jnp.float32),
            pltpu.VMEM((B_PER_W, EMBED), jnp.float32),
            pltpu.SemaphoreType.DMA,
        ],
    )
    def gather_mean(eidx_hbm, table_hbm, avg_hbm, idx_v, rows_v, avg_v, sem):
        wid = lax.axis_index("s") * NC + lax.axis_index("c")
        pltpu.sync_copy(eidx_hbm.at[wid], idx_v)

        def fire(g, carry):
            for k in range(FIRE):
                pltpu.async_copy(
                    table_hbm.at[idx_v.at[g * FIRE + k]],
                    rows_v.at[pl.ds((g * FIRE + k) * IDX_CHUNK, IDX_CHUNK)],
                    sem,
                )
            return carry

        lax.fori_loop(0, N_ROUNDS, fire, 0)
        pltpu.make_async_copy(
            table_hbm.at[pl.ds(0, ELEM_PER_W)], rows_v, sem
        ).wait()

        inv_ctx = jnp.float32(1.0 / CTX)

        def body(i, carry):
            base = i * (CTX * EMBED)
            acc = rows_v[pl.ds(base, EMBED)]
            for j in range(1, CTX):
                acc = acc + rows_v[pl.ds(base + j * EMBED, EMBED)]
            avg_v[i] = acc * inv_ctx
            return carry

        lax.fori_loop(0, B_PER_W, body, 0)
        pltpu.sync_copy(avg_v, avg_hbm.at[pl.ds(wid * B_PER_W, B_PER_W)])

    return gather_mean(eidx, table_flat)


VT = 4096


def _mm_body(w_ref, avgt_ref, b_ref, out_ref):
    out_ref[...] = (
        lax.dot_general(
            w_ref[...],
            avgt_ref[...],
            (((0,), (0,)), ((), ())),
            preferred_element_type=jnp.float32,
        )
        + b_ref[...][:, None]
    )


def _matmul_bias_tc(avg_t, fc_w, fc_b):
    nvt = (VOCAB + VT - 1) // VT
    out_t = pl.pallas_call(
        _mm_body,
        grid=(nvt,),
        in_specs=[
            pl.BlockSpec((EMBED, VT), lambda j: (0, j)),
            pl.BlockSpec((EMBED, BATCH), lambda j: (0, 0)),
            pl.BlockSpec((VT,), lambda j: (j,)),
        ],
        out_specs=pl.BlockSpec((VT, BATCH), lambda j: (j, 0)),
        out_shape=jax.ShapeDtypeStruct((VOCAB, BATCH), jnp.float32),
    )(fc_w, avg_t, fc_b)
    return jnp.transpose(out_t)


def kernel(x, emb_table, fc_w, fc_b):
    eidx = (
        x.reshape(-1, 1).astype(jnp.int32)
        + jnp.arange(EMBED, dtype=jnp.int32)[None, :] * VOCAB
    ).reshape(NW, N_CHUNKS, IDX_CHUNK)
    table_flat = emb_table.T.reshape(-1)
    avg = _gather_mean_sc(eidx, table_flat)
    return _matmul_bias_tc(avg.T, fc_w, fc_b)
